# baseline (speedup 1.0000x reference)
import functools

import jax
import jax.numpy as jnp
from jax import lax
from jax.experimental import pallas as pl
from jax.experimental.pallas import tpu as pltpu
from jax.experimental.pallas import tpu_sc as plsc

V_DIM = 100000
EMB_DIM = 64
BATCH = 4096
CTX = 50

NUM_CORES = 2
NUM_SUBCORES = 16
NW = NUM_CORES * NUM_SUBCORES
BPW = BATCH // NW
CHUNK = 16
NCHUNK = BPW // CHUNK
ROWS = CHUNK * CTX
LANES = 16
NGRP = EMB_DIM // (2 * LANES)
UNROLL = 10
SCALE = 1.0 / CTX

_mesh = plsc.VectorSubcoreMesh(core_axis_name="c", subcore_axis_name="s")


@functools.partial(
    pl.kernel,
    out_type=jax.ShapeDtypeStruct((BATCH, EMB_DIM), jnp.float32),
    mesh=_mesh,
    compiler_params=pltpu.CompilerParams(use_tc_tiling_on_sc=False, needs_layout_passes=False),
    scratch_types=[
        pltpu.VMEM((CTX, BPW), jnp.int32),
        pltpu.VMEM((NCHUNK, ROWS), jnp.int32),
        pltpu.VMEM((2, ROWS, EMB_DIM), jnp.bfloat16),
        pltpu.VMEM((BPW, EMB_DIM), jnp.float32),
        pltpu.SemaphoreType.DMA,
        pltpu.SemaphoreType.DMA,
    ],
)
def _cbow_sc(idx_hbm, table_hbm, out_hbm, idx_s, fl, rows_v, out_v, sem0, sem1):
    wid = lax.axis_index("s") * NUM_CORES + lax.axis_index("c")
    sems = (sem0, sem1)

    pltpu.sync_copy(idx_hbm.at[:, pl.ds(wid * BPW, BPW)], idx_s)

    for ch in range(NCHUNK):
        for c in range(CTX):
            fl[ch, pl.ds(c * CHUNK, CHUNK)] = idx_s[c, pl.ds(ch * CHUNK, CHUNK)]

    def start_gather(ch):
        buf = ch % 2
        return pltpu.async_copy(
            table_hbm.at[fl.at[ch]], rows_v.at[buf], sems[buf])

    handles = [start_gather(0)]
    for ch in range(NCHUNK):
        if ch + 1 < NCHUNK:
            handles.append(start_gather(ch + 1))
        handles[ch].wait()
        buf = ch % 2
        rv = rows_v.at[buf]

        def row_body(b, _, ch=ch, rv=rv):
            def body(t, acc):
                row0 = b + t * (UNROLL * CHUNK)
                acc = list(acc)
                for u in range(UNROLL):
                    for g in range(NGRP):
                        pair = rv[row0 + u * CHUNK, pl.ds(g * 2 * LANES, 2 * LANES)]
                        ev, od = plsc.unpack(
                            pair, format=plsc.PackFormat.INTERLEAVED,
                            preferred_element_type=jnp.float32)
                        bank = ((u % 2) * NGRP + g) * 2
                        acc[bank] = acc[bank] + ev
                        acc[bank + 1] = acc[bank + 1] + od
                return tuple(acc)

            acc0 = tuple(
                jnp.zeros((LANES,), jnp.float32) for _ in range(4 * NGRP))
            acc = lax.fori_loop(0, CTX // UNROLL, body, acc0)
            orow = ch * CHUNK + b
            rowv = jnp.full((LANES,), orow, jnp.int32)
            two = lax.iota(jnp.int32, LANES) * 2
            for g in range(NGRP):
                bank = g * 2
                ev = (acc[bank] + acc[bank + 2 * NGRP]) * SCALE
                od = (acc[bank + 1] + acc[bank + 1 + 2 * NGRP]) * SCALE
                plsc.store_scatter(out_v, [rowv, two + g * 2 * LANES], ev)
                plsc.store_scatter(out_v, [rowv, two + g * 2 * LANES + 1], od)
            return 0

        lax.fori_loop(0, CHUNK, row_body, 0)

    pltpu.sync_copy(out_v, out_hbm.at[pl.ds(wid * BPW, BPW)])


def kernel(x, embeddings):
    return _cbow_sc(x.T.astype(jnp.int32), embeddings.astype(jnp.bfloat16))


# speedup vs baseline: 1.1599x; 1.1599x over previous
import functools

import jax
import jax.numpy as jnp
from jax import lax
from jax.experimental import pallas as pl
from jax.experimental.pallas import tpu as pltpu
from jax.experimental.pallas import tpu_sc as plsc

V_DIM = 100000
EMB_DIM = 64
BATCH = 4096
CTX = 50

NUM_CORES = 2
NUM_SUBCORES = 16
NW = NUM_CORES * NUM_SUBCORES
BPW = BATCH // NW
CHUNK = 16
NCHUNK = BPW // CHUNK
ROWS = CHUNK * CTX
LANES = 16
NVREG = EMB_DIM // LANES
UNROLL = 10
SCALE = 1.0 / CTX

_mesh = plsc.VectorSubcoreMesh(core_axis_name="c", subcore_axis_name="s")


@functools.partial(
    pl.kernel,
    out_type=jax.ShapeDtypeStruct((BATCH, EMB_DIM), jnp.float32),
    mesh=_mesh,
    compiler_params=pltpu.CompilerParams(use_tc_tiling_on_sc=False),
    scratch_types=[
        pltpu.VMEM((CTX, BPW), jnp.int32),
        pltpu.VMEM((NCHUNK, ROWS), jnp.int32),
        pltpu.VMEM((NCHUNK, ROWS), jnp.int32),
        pltpu.VMEM((2, ROWS, EMB_DIM), jnp.float32),
        pltpu.VMEM((BPW, EMB_DIM), jnp.float32),
        pltpu.VMEM_SHARED((BPW, EMB_DIM), jnp.float32),
        pltpu.SemaphoreType.DMA,
        pltpu.SemaphoreType.DMA,
        pltpu.SemaphoreType.DMA,
    ],
)
def _cbow_sc(idx_hbm, table_hbm, out_hbm, idx_s, fl, dl, rows_v, out_v,
             acc_sh, sem0, sem1, asem):
    sid = lax.axis_index("s")
    wid = sid * NUM_CORES + lax.axis_index("c")
    sems = (sem0, sem1)

    pltpu.sync_copy(idx_hbm.at[:, pl.ds(wid * BPW, BPW)], idx_s)

    lane = lax.iota(jnp.int32, LANES)
    zero = jnp.zeros((LANES,), jnp.float32)
    for ch in range(NCHUNK):
        for c in range(CTX):
            fl[ch, pl.ds(c * CHUNK, CHUNK)] = idx_s[c, pl.ds(ch * CHUNK, CHUNK)]
        if ch % 2 == 1:
            dest = lane + ch * CHUNK
            for c in range(CTX):
                dl[ch, pl.ds(c * CHUNK, CHUNK)] = dest

    def start_gather(ch):
        buf = ch % 2
        return pltpu.async_copy(
            table_hbm.at[fl.at[ch]], rows_v.at[buf], sems[buf])

    zrows = rows_v.at[0].at[pl.ds(0, BPW)]
    for r in range(BPW):
        for k in range(NVREG):
            zrows[r, pl.ds(k * LANES, LANES)] = zero
    pltpu.sync_copy(zrows, acc_sh)

    handles = [start_gather(0)]
    add_handles = {}
    for ch in range(NCHUNK):
        if ch + 1 < NCHUNK:
            if ch >= 2 and (ch - 1) % 2 == 1:
                add_handles[ch - 1].wait()
            handles.append(start_gather(ch + 1))
        handles[ch].wait()
        buf = ch % 2
        rv = rows_v.at[buf]
        if ch % 2 == 1:
            add_handles[ch] = pltpu.async_copy(
                rv, acc_sh.at[dl.at[ch]], asem, add=True)
            continue

        def row_body(b, _, ch=ch, rv=rv):
            def body(t, acc):
                row0 = b + t * (UNROLL * CHUNK)
                acc = list(acc)
                for u in range(UNROLL):
                    for k in range(NVREG):
                        bank = (u % 2) * NVREG + k
                        acc[bank] = acc[bank] + rv[
                            row0 + u * CHUNK, pl.ds(k * LANES, LANES)]
                return tuple(acc)

            acc0 = tuple(
                jnp.zeros((LANES,), jnp.float32) for _ in range(2 * NVREG))
            acc = lax.fori_loop(0, CTX // UNROLL, body, acc0)
            orow = ch * CHUNK + b
            for k in range(NVREG):
                out_v[orow, pl.ds(k * LANES, LANES)] = (
                    acc[k] + acc[NVREG + k]) * SCALE
            return 0

        lax.fori_loop(0, CHUNK, row_body, 0)

    add_handles[NCHUNK - 1].wait()
    rb = rows_v.at[0].at[pl.ds(0, BPW)]
    pltpu.sync_copy(acc_sh, rb)
    for ch in range(1, NCHUNK, 2):
        for b in range(CHUNK):
            row = ch * CHUNK + b
            for k in range(NVREG):
                out_v[row, pl.ds(k * LANES, LANES)] = (
                    rb[row, pl.ds(k * LANES, LANES)] * SCALE)

    pltpu.sync_copy(out_v, out_hbm.at[pl.ds(wid * BPW, BPW)])


def kernel(x, embeddings):
    return _cbow_sc(x.T.astype(jnp.int32), embeddings)


# speedup vs baseline: 1.2013x; 1.0357x over previous
import functools

import jax
import jax.numpy as jnp
from jax import lax
from jax.experimental import pallas as pl
from jax.experimental.pallas import tpu as pltpu
from jax.experimental.pallas import tpu_sc as plsc

V_DIM = 100000
EMB_DIM = 64
BATCH = 4096
CTX = 50

NUM_CORES = 2
NUM_SUBCORES = 16
NW = NUM_CORES * NUM_SUBCORES
BPW = BATCH // NW
CHUNK = 16
NCHUNK = BPW // CHUNK
ROWS = CHUNK * CTX
LANES = 16
NVREG = EMB_DIM // LANES
UNROLL = 25
SCALE = 1.0 / CTX

_mesh = plsc.VectorSubcoreMesh(core_axis_name="c", subcore_axis_name="s")


@functools.partial(
    pl.kernel,
    out_type=jax.ShapeDtypeStruct((BATCH, EMB_DIM), jnp.float32),
    mesh=_mesh,
    compiler_params=pltpu.CompilerParams(use_tc_tiling_on_sc=False),
    scratch_types=[
        pltpu.VMEM((CTX, BPW), jnp.int32),
        pltpu.VMEM((NCHUNK, ROWS), jnp.int32),
        pltpu.VMEM((2, ROWS, EMB_DIM), jnp.float32),
        pltpu.VMEM((BPW, EMB_DIM), jnp.float32),
        pltpu.SemaphoreType.DMA,
        pltpu.SemaphoreType.DMA,
    ],
)
def _cbow_sc(idx_hbm, table_hbm, out_hbm, idx_s, fl, rows_v, out_v, sem0, sem1):
    wid = lax.axis_index("s") * NUM_CORES + lax.axis_index("c")
    sems = (sem0, sem1)

    pltpu.sync_copy(idx_hbm.at[:, pl.ds(wid * BPW, BPW)], idx_s)

    for ch in range(NCHUNK):
        for c in range(CTX):
            fl[ch, pl.ds(c * CHUNK, CHUNK)] = idx_s[c, pl.ds(ch * CHUNK, CHUNK)]

    def start_gather(ch):
        buf = ch % 2
        return pltpu.async_copy(
            table_hbm.at[fl.at[ch]], rows_v.at[buf], sems[buf])

    handles = [start_gather(0)]
    for ch in range(NCHUNK):
        if ch + 1 < NCHUNK:
            handles.append(start_gather(ch + 1))
        handles[ch].wait()
        buf = ch % 2
        rv = rows_v.at[buf]

        def row_body(b, _, ch=ch, rv=rv):
            def body(t, acc):
                row0 = b + t * (UNROLL * CHUNK)
                acc = list(acc)
                for u in range(UNROLL):
                    for k in range(NVREG):
                        bank = (u % 2) * NVREG + k
                        acc[bank] = acc[bank] + rv[
                            row0 + u * CHUNK, pl.ds(k * LANES, LANES)]
                return tuple(acc)

            acc0 = tuple(
                jnp.zeros((LANES,), jnp.float32) for _ in range(2 * NVREG))
            acc = lax.fori_loop(0, CTX // UNROLL, body, acc0)
            orow = ch * CHUNK + b
            for k in range(NVREG):
                out_v[orow, pl.ds(k * LANES, LANES)] = (
                    acc[k] + acc[NVREG + k]) * SCALE
            return 0

        lax.fori_loop(0, CHUNK, row_body, 0)

    pltpu.sync_copy(out_v, out_hbm.at[pl.ds(wid * BPW, BPW)])


def kernel(x, embeddings):
    return _cbow_sc(x.T.astype(jnp.int32), embeddings)


# speedup vs baseline: 1.2173x; 1.0133x over previous
import functools

import jax
import jax.numpy as jnp
from jax import lax
from jax.experimental import pallas as pl
from jax.experimental.pallas import tpu as pltpu
from jax.experimental.pallas import tpu_sc as plsc

V_DIM = 100000
EMB_DIM = 64
BATCH = 4096
CTX = 50

NUM_CORES = 2
NUM_SUBCORES = 16
NW = NUM_CORES * NUM_SUBCORES
BPW = BATCH // NW
CHUNK = 16
NCHUNK = BPW // CHUNK
ROWS = CHUNK * CTX
LANES = 16
NVREG = EMB_DIM // LANES
UNROLL = 10
SCALE = 1.0 / CTX

_mesh = plsc.VectorSubcoreMesh(core_axis_name="c", subcore_axis_name="s")


@functools.partial(
    pl.kernel,
    out_type=jax.ShapeDtypeStruct((BATCH, EMB_DIM), jnp.float32),
    mesh=_mesh,
    compiler_params=pltpu.CompilerParams(use_tc_tiling_on_sc=False),
    scratch_types=[
        pltpu.VMEM((NCHUNK, ROWS), jnp.int32),
        pltpu.VMEM((2, ROWS, EMB_DIM), jnp.float32),
        pltpu.VMEM((BPW, EMB_DIM), jnp.float32),
        pltpu.SemaphoreType.DMA,
        pltpu.SemaphoreType.DMA,
    ],
)
def _cbow_sc(idx_hbm, table_hbm, out_hbm, idx_v, rows_v, out_v, sem0, sem1):
    wid = lax.axis_index("s") * NUM_CORES + lax.axis_index("c")
    sems = (sem0, sem1)

    pltpu.sync_copy(idx_hbm.at[wid], idx_v)

    def start_gather(ch):
        buf = ch % 2
        return pltpu.async_copy(
            table_hbm.at[idx_v.at[ch]], rows_v.at[buf], sems[buf])

    handles = [start_gather(0)]
    for ch in range(NCHUNK):
        if ch + 1 < NCHUNK:
            handles.append(start_gather(ch + 1))
        handles[ch].wait()
        buf = ch % 2
        rv = rows_v.at[buf]

        def row_body(r, _, ch=ch, rv=rv):
            base = r * CTX

            def body(t, acc):
                row0 = base + t * UNROLL
                acc = list(acc)
                for u in range(UNROLL):
                    for k in range(NVREG):
                        bank = (u % 2) * NVREG + k
                        acc[bank] = acc[bank] + rv[
                            row0 + u, pl.ds(k * LANES, LANES)]
                return tuple(acc)

            acc0 = tuple(
                jnp.zeros((LANES,), jnp.float32) for _ in range(2 * NVREG))
            acc = lax.fori_loop(0, CTX // UNROLL, body, acc0)
            orow = ch * CHUNK + r
            for k in range(NVREG):
                out_v[orow, pl.ds(k * LANES, LANES)] = (
                    acc[k] + acc[NVREG + k]) * SCALE
            return 0

        lax.fori_loop(0, CHUNK, row_body, 0)

    pltpu.sync_copy(out_v, out_hbm.at[pl.ds(wid * BPW, BPW)])


def kernel(x, embeddings):
    idx = x.astype(jnp.int32).reshape(NW, NCHUNK, ROWS)
    return _cbow_sc(idx, embeddings)
